# baseline (speedup 1.0000x reference)
import jax
import jax.numpy as jnp
from jax.experimental import pallas as pl

_N = 10000
_F_IN = 128
_HID = 32
_NC = 2
_BLK = 1000


def _cell_body(x_ref, h_ref, c_ref,
               Wi_ref, Wf_ref, Wc_ref, Wo_ref,
               Thi_ref, Thf_ref, Thc_ref, Tho_ref,
               bi_ref, bf_ref, bc_ref, bo_ref,
               wci_ref, wcf_ref, wco_ref, Wlin_ref, blin_ref,
               logits_ref, H_ref, C_ref):
    xb = x_ref[...]
    hb = h_ref[...]
    cb = c_ref[...]
    f32 = jnp.float32
    zi = (jnp.dot(xb, Wi_ref[...], preferred_element_type=f32)
          + jnp.dot(hb, Thi_ref[...], preferred_element_type=f32)
          + wci_ref[...] * cb + bi_ref[...])
    zf = (jnp.dot(xb, Wf_ref[...], preferred_element_type=f32)
          + jnp.dot(hb, Thf_ref[...], preferred_element_type=f32)
          + wcf_ref[...] * cb + bf_ref[...])
    zc = (jnp.dot(xb, Wc_ref[...], preferred_element_type=f32)
          + jnp.dot(hb, Thc_ref[...], preferred_element_type=f32)
          + bc_ref[...])
    zo = (jnp.dot(xb, Wo_ref[...], preferred_element_type=f32)
          + jnp.dot(hb, Tho_ref[...], preferred_element_type=f32)
          + bo_ref[...])
    ig = 0.5 * jnp.tanh(zi) + 0.5
    fg = 0.5 * jnp.tanh(zf) + 0.5
    tg = jnp.tanh(zc)
    Cn = fg * cb + ig * tg
    og = 0.5 * jnp.tanh(zo + wco_ref[...] * Cn) + 0.5
    Hn = og * jnp.tanh(Cn)
    C_ref[...] = Cn
    H_ref[...] = Hn
    logits_ref[...] = (
        jnp.dot(jnp.maximum(Hn, 0.0), Wlin_ref[...],
                preferred_element_type=f32)
        + blin_ref[...])


def kernel(x, edge_index, edge_weight, h, c,
           W_i, Th_i, cb_i, w_c_i, b_i,
           W_f, Th_f, cb_f, w_c_f, b_f,
           W_c, Th_c, cb_c, b_c,
           W_o, Th_o, cb_o, w_c_o, b_o,
           W_lin, b_lin):
    del edge_index, edge_weight
    half = jnp.float32(0.5)
    Wi, Thi, bi, wci = half * W_i, half * Th_i, half * (cb_i[None, :] + b_i), half * w_c_i
    Wf, Thf, bf, wcf = half * W_f, half * Th_f, half * (cb_f[None, :] + b_f), half * w_c_f
    Wo, Tho, bo, wco = half * W_o, half * Th_o, half * (cb_o[None, :] + b_o), half * w_c_o
    bc = cb_c[None, :] + b_c
    blin = b_lin[None, :]

    grid = (_N // _BLK,)
    row_spec = lambda w: pl.BlockSpec((_BLK, w), lambda i: (i, 0))
    full_spec = lambda s: pl.BlockSpec(s, lambda i: (0, 0))

    logits, H, C = pl.pallas_call(
        _cell_body,
        grid=grid,
        in_specs=[
            row_spec(_F_IN),
            row_spec(_HID),
            row_spec(_HID),
            full_spec((_F_IN, _HID)),
            full_spec((_F_IN, _HID)),
            full_spec((_F_IN, _HID)),
            full_spec((_F_IN, _HID)),
            full_spec((_HID, _HID)),
            full_spec((_HID, _HID)),
            full_spec((_HID, _HID)),
            full_spec((_HID, _HID)),
            full_spec((1, _HID)),
            full_spec((1, _HID)),
            full_spec((1, _HID)),
            full_spec((1, _HID)),
            full_spec((1, _HID)),
            full_spec((1, _HID)),
            full_spec((1, _HID)),
            full_spec((_HID, _NC)),
            full_spec((1, _NC)),
        ],
        out_specs=[
            row_spec(_NC),
            row_spec(_HID),
            row_spec(_HID),
        ],
        out_shape=[
            jax.ShapeDtypeStruct((_N, _NC), jnp.float32),
            jax.ShapeDtypeStruct((_N, _HID), jnp.float32),
            jax.ShapeDtypeStruct((_N, _HID), jnp.float32),
        ],
    )(x, h, c, Wi, Wf, W_c, Wo, Thi, Thf, Th_c, Tho,
      bi, bf, bc, bo, wci, wcf, wco, W_lin, blin)
    return (logits, H, C)


# speedup vs baseline: 2.6040x; 2.6040x over previous
import jax
import jax.numpy as jnp
from jax.experimental import pallas as pl
from jax.experimental.pallas import tpu as pltpu

_N = 10000
_F_IN = 128
_HID = 32
_NC = 2


def _cell_body(x_ref, hT_ref, cT_ref, WT_ref, ThT_ref, bT_ref,
               wciT_ref, wcfT_ref, wcoT_ref, WlinT_ref, blinT_ref,
               logitsT_ref, HT_ref, CT_ref):
    f32 = jnp.float32
    xb = x_ref[...]
    hTb = hT_ref[...]
    cTb = cT_ref[...]
    gT = jax.lax.dot_general(WT_ref[...], xb, (((1,), (1,)), ((), ())),
                             preferred_element_type=f32)
    gT = gT + jnp.dot(ThT_ref[...], hTb, preferred_element_type=f32)
    gT = gT + bT_ref[...]
    iT = 0.5 * jnp.tanh(gT[0:_HID, :] + wciT_ref[...] * cTb) + 0.5
    fT = 0.5 * jnp.tanh(gT[_HID:2 * _HID, :] + wcfT_ref[...] * cTb) + 0.5
    tT = jnp.tanh(gT[2 * _HID:3 * _HID, :])
    Cn = fT * cTb + iT * tT
    oT = 0.5 * jnp.tanh(gT[3 * _HID:4 * _HID, :] + wcoT_ref[...] * Cn) + 0.5
    Hn = oT * jnp.tanh(Cn)
    CT_ref[...] = Cn
    HT_ref[...] = Hn
    logitsT_ref[...] = (
        jnp.dot(WlinT_ref[...], jnp.maximum(Hn, 0.0),
                preferred_element_type=f32)
        + blinT_ref[...])


def kernel(x, edge_index, edge_weight, h, c,
           W_i, Th_i, cb_i, w_c_i, b_i,
           W_f, Th_f, cb_f, w_c_f, b_f,
           W_c, Th_c, cb_c, b_c,
           W_o, Th_o, cb_o, w_c_o, b_o,
           W_lin, b_lin):
    del edge_index, edge_weight
    half = jnp.float32(0.5)
    WT = jnp.concatenate([half * W_i.T, half * W_f.T,
                          W_c.T, half * W_o.T], axis=0)
    ThT = jnp.concatenate([half * Th_i.T, half * Th_f.T,
                           Th_c.T, half * Th_o.T], axis=0)
    bT = jnp.concatenate([half * (cb_i[None, :] + b_i),
                          half * (cb_f[None, :] + b_f),
                          cb_c[None, :] + b_c,
                          half * (cb_o[None, :] + b_o)], axis=1).T
    wciT, wcfT, wcoT = (half * w_c_i).T, (half * w_c_f).T, (half * w_c_o).T
    WlinT = W_lin.T
    blinT = b_lin[:, None]

    vmem = pl.BlockSpec(memory_space=pltpu.VMEM)
    logitsT, HT, CT = pl.pallas_call(
        _cell_body,
        in_specs=[vmem] * 11,
        out_specs=[vmem, vmem, vmem],
        out_shape=[
            jax.ShapeDtypeStruct((_NC, _N), jnp.float32),
            jax.ShapeDtypeStruct((_HID, _N), jnp.float32),
            jax.ShapeDtypeStruct((_HID, _N), jnp.float32),
        ],
    )(x, h.T, c.T, WT, ThT, bT, wciT, wcfT, wcoT, WlinT, blinT)
    return (logitsT.T, HT.T, CT.T)
